# Initial kernel scaffold; baseline (speedup 1.0000x reference)
import functools

import jax
import jax.numpy as jnp
from jax import lax
from jax.experimental import pallas as pl
from jax.experimental.pallas import tpu as pltpu
from jax.experimental.pallas import tpu_sc as plsc

_LN_EPS = 1e-12
_GATHER_WINDOW = 32


def _sc_gather(table, ids):
    n = ids.shape[0]
    h = table.shape[1]
    ids2 = ids.reshape(1, n)
    mesh = plsc.VectorSubcoreMesh(core_axis_name="c", subcore_axis_name="s")

    @functools.partial(
        pl.kernel,
        out_type=jax.ShapeDtypeStruct((n, h), table.dtype),
        mesh=mesh,
    )
    def k(tab_hbm, idx_hbm, out_hbm):
        def body(i_vmem, o_vmem):
            pltpu.sync_copy(tab_hbm.at[i_vmem.at[0]], o_vmem)

        pltpu.emit_pipeline(
            body,
            grid=(n // _GATHER_WINDOW,),
            in_specs=[
                pl.BlockSpec((1, _GATHER_WINDOW), lambda i: (0, i)),
            ],
            out_specs=[
                pl.BlockSpec((_GATHER_WINDOW, h), lambda i: (i, 0)),
            ],
            core_axis_name=("c", "s"),
            dimension_semantics=(pltpu.PARALLEL,),
        )(idx_hbm, out_hbm)

    return k(table, ids2)


def _tc_ln_body(g_ref, p_ref, t_ref, tab_ref, gam_ref, bet_ref, o_ref):
    tt = t_ref[0, 0, :].astype(jnp.float32)[:, None]
    base = tab_ref[0:1, :]
    diff = tab_ref[1:2, :] - tab_ref[0:1, :]
    x = g_ref[...] + p_ref[...] + base + tt * diff
    mean = jnp.mean(x, axis=1, keepdims=True)
    xc = x - mean
    var = jnp.mean(xc * xc, axis=1, keepdims=True)
    y = xc * lax.rsqrt(var + _LN_EPS)
    o_ref[...] = y * gam_ref[...] + bet_ref[...]


def _tc_ln(gathered, pos_emb, type_ids3, type_tab, gamma2, beta2, block):
    n, h = gathered.shape
    s = pos_emb.shape[0]
    b = n // s
    sb = s // block
    grid = (b, sb)
    return pl.pallas_call(
        _tc_ln_body,
        grid=grid,
        in_specs=[
            pl.BlockSpec((block, h), lambda i, j: (i * sb + j, 0)),
            pl.BlockSpec((block, h), lambda i, j: (j, 0)),
            pl.BlockSpec((1, 1, block), lambda i, j: (i, 0, j)),
            pl.BlockSpec((2, h), lambda i, j: (0, 0)),
            pl.BlockSpec((1, h), lambda i, j: (0, 0)),
            pl.BlockSpec((1, h), lambda i, j: (0, 0)),
        ],
        out_specs=pl.BlockSpec((block, h), lambda i, j: (i * sb + j, 0)),
        out_shape=jax.ShapeDtypeStruct((n, h), jnp.float32),
        compiler_params=pltpu.CompilerParams(
            dimension_semantics=("parallel", "parallel"),
        ),
    )(gathered, pos_emb, type_ids3, type_tab, gamma2, beta2)


def kernel(input_ids, token_type_ids, word_embeddings, position_embeddings,
           token_type_embeddings, ln_gamma, ln_beta):
    b, s = input_ids.shape
    h = word_embeddings.shape[1]
    ids = input_ids.reshape(-1).astype(jnp.int32)
    gathered = _sc_gather(word_embeddings, ids)
    out = _tc_ln(
        gathered,
        position_embeddings[:s],
        token_type_ids.reshape(b, 1, s).astype(jnp.int32),
        token_type_embeddings,
        ln_gamma.reshape(1, h),
        ln_beta.reshape(1, h),
        block=256,
    )
    return out.reshape(b, s, h)


# speedup vs baseline: 1.4231x; 1.4231x over previous
import functools

import jax
import jax.numpy as jnp
from jax import lax
from jax.experimental import pallas as pl
from jax.experimental.pallas import tpu as pltpu
from jax.experimental.pallas import tpu_sc as plsc

_LN_EPS = 1e-12
_NUM_WORKERS = 32
_GATHER_CHUNK = 64


def _sc_gather(table, ids):
    n = ids.shape[0]
    h = table.shape[1]
    b_per_w = n // _NUM_WORKERS
    nchunk = b_per_w // _GATHER_CHUNK
    mesh = plsc.VectorSubcoreMesh(core_axis_name="c", subcore_axis_name="s")

    @functools.partial(
        pl.kernel,
        out_type=jax.ShapeDtypeStruct((n, h), table.dtype),
        mesh=mesh,
        scratch_types=[
            pltpu.VMEM((b_per_w,), jnp.int32),
            pltpu.VMEM((_GATHER_CHUNK, h), table.dtype),
            pltpu.SemaphoreType.DMA,
        ],
    )
    def k(tab_hbm, idx_hbm, out_hbm, idx_v, rows_v, sem):
        wid = lax.axis_index("s") * 2 + lax.axis_index("c")
        base = wid * b_per_w
        pltpu.sync_copy(idx_hbm.at[pl.ds(base, b_per_w)], idx_v)

        @pl.loop(0, nchunk)
        def _(j):
            off = j * _GATHER_CHUNK
            pltpu.async_copy(
                tab_hbm.at[idx_v.at[pl.ds(off, _GATHER_CHUNK)]], rows_v, sem
            ).wait()
            pltpu.sync_copy(
                rows_v, out_hbm.at[pl.ds(base + off, _GATHER_CHUNK)]
            )

    return k(table, ids)


def _tc_ln_body(g_ref, p_ref, t_ref, tab_ref, gam_ref, bet_ref, o_ref):
    tt = t_ref[0, 0, :].astype(jnp.float32)[:, None]
    base = tab_ref[0:1, :]
    diff = tab_ref[1:2, :] - tab_ref[0:1, :]
    x = g_ref[...] + p_ref[...] + base + tt * diff
    mean = jnp.mean(x, axis=1, keepdims=True)
    xc = x - mean
    var = jnp.mean(xc * xc, axis=1, keepdims=True)
    y = xc * lax.rsqrt(var + _LN_EPS)
    o_ref[...] = y * gam_ref[...] + bet_ref[...]


def _tc_ln(gathered, pos_emb, type_ids3, type_tab, gamma2, beta2, block):
    n, h = gathered.shape
    s = pos_emb.shape[0]
    b = n // s
    sb = s // block
    grid = (b, sb)
    return pl.pallas_call(
        _tc_ln_body,
        grid=grid,
        in_specs=[
            pl.BlockSpec((block, h), lambda i, j: (i * sb + j, 0)),
            pl.BlockSpec((block, h), lambda i, j: (j, 0)),
            pl.BlockSpec((1, 1, block), lambda i, j: (i, 0, j)),
            pl.BlockSpec((2, h), lambda i, j: (0, 0)),
            pl.BlockSpec((1, h), lambda i, j: (0, 0)),
            pl.BlockSpec((1, h), lambda i, j: (0, 0)),
        ],
        out_specs=pl.BlockSpec((block, h), lambda i, j: (i * sb + j, 0)),
        out_shape=jax.ShapeDtypeStruct((n, h), jnp.float32),
        compiler_params=pltpu.CompilerParams(
            dimension_semantics=("parallel", "parallel"),
        ),
    )(gathered, pos_emb, type_ids3, type_tab, gamma2, beta2)


def kernel(input_ids, token_type_ids, word_embeddings, position_embeddings,
           token_type_embeddings, ln_gamma, ln_beta):
    b, s = input_ids.shape
    h = word_embeddings.shape[1]
    ids = input_ids.reshape(-1).astype(jnp.int32)
    gathered = _sc_gather(word_embeddings, ids)
    out = _tc_ln(
        gathered,
        position_embeddings[:s],
        token_type_ids.reshape(b, 1, s).astype(jnp.int32),
        token_type_embeddings,
        ln_gamma.reshape(1, h),
        ln_beta.reshape(1, h),
        block=256,
    )
    return out.reshape(b, s, h)
